# speedup vs baseline: 1.4692x; 1.4692x over previous
import jax
import jax.numpy as jnp
from jax import lax
from jax.experimental import pallas as pl
from jax.experimental.pallas import tpu as pltpu
from jax.experimental.pallas import tpu_sc as plsc

_INFO = plsc.get_sparse_core_info()
_NC = _INFO.num_cores
_NS = _INFO.num_subcores
_NW = _NC * _NS

_B = 4096 * 8
_D = 256
_BPW = _B // _NW
_C = 128
_NCHUNK = _BPW // _C


def _gather_body(x_hbm, idx_hbm, out_hbm, idx_v, buf0, buf1, gsem0, gsem1,
                 osem0, osem1):
  wid = lax.axis_index("s") * _NC + lax.axis_index("c")
  base = wid * _BPW

  pltpu.sync_copy(idx_hbm.at[pl.ds(base, _BPW)], idx_v)

  bufs = (buf0, buf1)
  gsems = (gsem0, gsem1)
  osems = (osem0, osem1)

  def start_gather(g):
    b = g % 2
    return pltpu.async_copy(
        x_hbm.at[idx_v.at[pl.ds(g * _C, _C)]], bufs[b], gsems[b])

  gathers = [None] * _NCHUNK
  writes = [None] * _NCHUNK
  gathers[0] = start_gather(0)
  for g in range(_NCHUNK):
    b = g % 2
    gathers[g].wait()
    if g + 1 < _NCHUNK:
      if g >= 1:
        writes[g - 1].wait()
      gathers[g + 1] = start_gather(g + 1)
    writes[g] = pltpu.async_copy(
        bufs[b], out_hbm.at[pl.ds(base + g * _C, _C)], osems[b])
  writes[_NCHUNK - 2].wait()
  writes[_NCHUNK - 1].wait()


@jax.jit
def _pooled_gather(x, idx):
  mesh = plsc.VectorSubcoreMesh(core_axis_name="c", subcore_axis_name="s")
  return pl.kernel(
      _gather_body,
      out_type=jax.ShapeDtypeStruct((_B, _D), jnp.float32),
      mesh=mesh,
      scratch_types=[
          pltpu.VMEM((_BPW,), jnp.int32),
          pltpu.VMEM((_C, _D), jnp.float32),
          pltpu.VMEM((_C, _D), jnp.float32),
          pltpu.SemaphoreType.DMA,
          pltpu.SemaphoreType.DMA,
          pltpu.SemaphoreType.DMA,
          pltpu.SemaphoreType.DMA,
      ],
  )(x, idx)


def kernel(x, pooling_mask):
  return _pooled_gather(x, pooling_mask.reshape(-1))


# speedup vs baseline: 1.5646x; 1.0650x over previous
import jax
import jax.numpy as jnp
from jax import lax
from jax.experimental import pallas as pl
from jax.experimental.pallas import tpu as pltpu
from jax.experimental.pallas import tpu_sc as plsc

_INFO = plsc.get_sparse_core_info()
_NC = _INFO.num_cores
_NS = _INFO.num_subcores
_NW = _NC * _NS

_B = 4096 * 8
_D = 256
_BPW = _B // _NW
_C = 128
_NCHUNK = _BPW // _C


_NBUF = 3


def _gather_body(x_hbm, idx_hbm, out_hbm, idx_v, buf0, buf1, buf2, gsem0,
                 gsem1, gsem2, osem0, osem1, osem2):
  wid = lax.axis_index("s") * _NC + lax.axis_index("c")
  base = wid * _BPW

  pltpu.sync_copy(idx_hbm.at[pl.ds(base, _BPW)], idx_v)

  bufs = (buf0, buf1, buf2)
  gsems = (gsem0, gsem1, gsem2)
  osems = (osem0, osem1, osem2)

  def start_gather(g):
    b = g % _NBUF
    return pltpu.async_copy(
        x_hbm.at[idx_v.at[pl.ds(g * _C, _C)]], bufs[b], gsems[b])

  gathers = [None] * _NCHUNK
  writes = [None] * _NCHUNK
  for g in range(min(_NBUF, _NCHUNK)):
    gathers[g] = start_gather(g)
  for g in range(_NCHUNK):
    b = g % _NBUF
    gathers[g].wait()
    writes[g] = pltpu.async_copy(
        bufs[b], out_hbm.at[pl.ds(base + g * _C, _C)], osems[b])
    if g + _NBUF < _NCHUNK:
      writes[g].wait()
      gathers[g + _NBUF] = start_gather(g + _NBUF)
  for g in range(max(0, _NCHUNK - _NBUF), _NCHUNK):
    writes[g].wait()


@jax.jit
def _pooled_gather(x, idx):
  mesh = plsc.VectorSubcoreMesh(core_axis_name="c", subcore_axis_name="s")
  return pl.kernel(
      _gather_body,
      out_type=jax.ShapeDtypeStruct((_B, _D), jnp.float32),
      mesh=mesh,
      scratch_types=(
          [pltpu.VMEM((_BPW,), jnp.int32)]
          + [pltpu.VMEM((_C, _D), jnp.float32)] * _NBUF
          + [pltpu.SemaphoreType.DMA] * (2 * _NBUF)
      ),
  )(x, idx)


def kernel(x, pooling_mask):
  return _pooled_gather(x, pooling_mask.reshape(-1))


# speedup vs baseline: 1.5678x; 1.0020x over previous
import jax
import jax.numpy as jnp
from jax import lax
from jax.experimental import pallas as pl
from jax.experimental.pallas import tpu as pltpu
from jax.experimental.pallas import tpu_sc as plsc

_INFO = plsc.get_sparse_core_info()
_NC = _INFO.num_cores
_NS = _INFO.num_subcores
_NW = _NC * _NS

_B = 4096 * 8
_D = 256
_BPW = _B // _NW
_C = 64
_NCHUNK = _BPW // _C
_NBUF = 6
_LAG = 4


def _gather_body(x_hbm, idx_hbm, out_hbm, idx_v, *rest):
  bufs = rest[:_NBUF]
  gsems = rest[_NBUF:2 * _NBUF]
  osems = rest[2 * _NBUF:]

  wid = lax.axis_index("s") * _NC + lax.axis_index("c")
  base = wid * _BPW

  pltpu.sync_copy(idx_hbm.at[pl.ds(base, _BPW)], idx_v)

  def start_gather(g):
    b = g % _NBUF
    return pltpu.async_copy(
        x_hbm.at[idx_v.at[pl.ds(g * _C, _C)]], bufs[b], gsems[b])

  gathers = [None] * _NCHUNK
  writes = [None] * _NCHUNK
  for g in range(min(_LAG, _NCHUNK)):
    gathers[g] = start_gather(g)
  for g in range(_NCHUNK):
    gathers[g].wait()
    writes[g] = pltpu.async_copy(
        bufs[g % _NBUF], out_hbm.at[pl.ds(base + g * _C, _C)],
        osems[g % _NBUF])
    h = g + _LAG
    if h < _NCHUNK:
      if h - _NBUF >= 0:
        writes[h - _NBUF].wait()
      gathers[h] = start_gather(h)
  for g in range(max(0, _NCHUNK - _NBUF), _NCHUNK):
    writes[g].wait()


@jax.jit
def _pooled_gather(x, idx):
  mesh = plsc.VectorSubcoreMesh(core_axis_name="c", subcore_axis_name="s")
  return pl.kernel(
      _gather_body,
      out_type=jax.ShapeDtypeStruct((_B, _D), jnp.float32),
      mesh=mesh,
      scratch_types=(
          [pltpu.VMEM((_BPW,), jnp.int32)]
          + [pltpu.VMEM((_C, _D), jnp.float32)] * _NBUF
          + [pltpu.SemaphoreType.DMA] * (2 * _NBUF)
      ),
  )(x, idx)


def kernel(x, pooling_mask):
  return _pooled_gather(x, pooling_mask.reshape(-1))
